# baseline (speedup 1.0000x reference)
import functools

import jax
import jax.numpy as jnp
from jax import lax
from jax.experimental import pallas as pl
from jax.experimental.pallas import tpu as pltpu
from jax.experimental.pallas import tpu_sc as plsc

_NUM_DRIVES = 64
_DIM = 128
_LOC_DIM = 64
_M = 100000
_K = 5
_THRESH = 0.1
_TILE = 10000
_GRID = _M // _TILE
def _topk_body(q_ref, s_ref, idx_ref, nf_ref, vals_s, idx_s):
    i = pl.program_id(0)
    neg = jnp.float32(-jnp.inf)
    _BIG = jnp.int32(2**31 - 1)

    q = q_ref[...]
    qn = q / (jnp.sqrt(jnp.sum(q * q, axis=1, keepdims=True)) + 1e-8)
    s = s_ref[...]
    nrm = jnp.sqrt(jnp.sum(s * s, axis=1, keepdims=True)) + 1e-8
    sn = (s / nrm).astype(jnp.bfloat16)
    qb = qn.astype(jnp.bfloat16)
    scores = lax.dot_general(qb, sn, (((1,), (1,)), ((), ())),
                             preferred_element_type=jnp.float32)

    col = lax.broadcasted_iota(jnp.int32, (_NUM_DRIVES, _TILE), 1) + i * _TILE
    cur = scores
    tv, ti = [], []
    for _ in range(_K):
        m = jnp.max(cur, axis=1, keepdims=True)
        sel = jnp.min(jnp.where(cur == m, col, _BIG), axis=1, keepdims=True)
        tv.append(m)
        ti.append(sel)
        cur = jnp.where(col == sel, neg, cur)
    tvc = jnp.concatenate(tv, axis=1)
    tic = jnp.concatenate(ti, axis=1)

    @pl.when(i == 0)
    def _():
        vals_s[...] = jnp.full((_NUM_DRIVES, _K), neg, jnp.float32)
        idx_s[...] = jnp.full((_NUM_DRIVES, _K), 2**31 - 1, jnp.int32)

    cv = jnp.concatenate([vals_s[...], tvc], axis=1)
    ci = jnp.concatenate([idx_s[...], tic], axis=1)
    ov, oi = [], []
    for _ in range(_K):
        m = jnp.max(cv, axis=1, keepdims=True)
        sel = jnp.min(jnp.where(cv == m, ci, _BIG), axis=1, keepdims=True)
        ov.append(m)
        oi.append(sel)
        cv = jnp.where(ci == sel, neg, cv)
    nv = jnp.concatenate(ov, axis=1)
    ni = jnp.concatenate(oi, axis=1)
    vals_s[...] = nv
    idx_s[...] = ni

    @pl.when(i == _GRID - 1)
    def _():
        idx_ref[...] = ni
        nf_ref[...] = jnp.sum((nv > _THRESH).astype(jnp.int32), axis=1,
                              keepdims=True)


def _topk(drive_emb, mem_senses):
    return pl.pallas_call(
        _topk_body,
        grid=(_GRID,),
        in_specs=[
            pl.BlockSpec((_NUM_DRIVES, _DIM), lambda i: (0, 0)),
            pl.BlockSpec((_TILE, _DIM), lambda i: (i, 0)),
        ],
        out_specs=[
            pl.BlockSpec((_NUM_DRIVES, _K), lambda i: (0, 0)),
            pl.BlockSpec((_NUM_DRIVES, 1), lambda i: (0, 0)),
        ],
        out_shape=[
            jax.ShapeDtypeStruct((_NUM_DRIVES, _K), jnp.int32),
            jax.ShapeDtypeStruct((_NUM_DRIVES, 1), jnp.int32),
        ],
        scratch_shapes=[
            pltpu.VMEM((_NUM_DRIVES, _K), jnp.float32),
            pltpu.VMEM((_NUM_DRIVES, _K), jnp.int32),
        ],
    )(drive_emb, mem_senses)


_N_ROWS = _NUM_DRIVES * _K
_N_WORKERS = 8
_ROWS_PER = _N_ROWS // _N_WORKERS


def _gather_senses(idx_flat, mem_senses):
    info = plsc.get_sparse_core_info()
    nc = info.num_cores

    mesh = plsc.VectorSubcoreMesh(core_axis_name="c", subcore_axis_name="s")

    @functools.partial(
        pl.kernel,
        mesh=mesh,
        out_type=jax.ShapeDtypeStruct((_N_ROWS, _DIM), jnp.float32),
        scratch_types=[
            pltpu.VMEM((_ROWS_PER,), jnp.int32),
            pltpu.VMEM((_ROWS_PER, _DIM), jnp.float32),
            pltpu.SemaphoreType.DMA,
        ],
    )
    def gk(idx_hbm, sen_hbm, out_sen, idx_v, sen_v, sem):
        wid = lax.axis_index("s") * nc + lax.axis_index("c")

        @pl.when(wid < _N_WORKERS)
        def _():
            base = pl.multiple_of(wid * _ROWS_PER, 8)
            pltpu.sync_copy(idx_hbm.at[pl.ds(base, _ROWS_PER)], idx_v)
            pltpu.async_copy(sen_hbm.at[idx_v], sen_v, sem).wait()
            pltpu.sync_copy(sen_v, out_sen.at[pl.ds(base, _ROWS_PER)])

    return gk(idx_flat, mem_senses)


_IPS = 32
_GSTEPS = _N_ROWS // _IPS


def _locsds_body(idx_ref, *refs):
    loc_blks = refs[:_IPS]
    sds_blks = refs[_IPS:2 * _IPS]
    out_loc, out_sds = refs[2 * _IPS], refs[2 * _IPS + 1]
    i = pl.program_id(0)
    dn = (((1,), (1,)), ((), ()))
    targets = jnp.concatenate(
        [jnp.full((1, 1), 128 * k, jnp.int32) + idx_ref[i * _IPS + k] % 128
         for k in range(_IPS)], axis=0)
    lanes = lax.broadcasted_iota(jnp.int32, (_IPS, 128 * _IPS), 1)
    e = (lanes == targets).astype(jnp.float32)
    loc_cat = jnp.concatenate([b[...] for b in loc_blks], axis=1)
    sds_cat = jnp.concatenate([b[...] for b in sds_blks], axis=1)
    out_loc[...] = lax.dot_general(e, loc_cat, dn,
                                   precision=lax.Precision.HIGHEST,
                                   preferred_element_type=jnp.float32)
    out_sds[...] = lax.dot_general(e, sds_cat, dn,
                                   precision=lax.Precision.HIGHEST,
                                   preferred_element_type=jnp.float32)


def _gather_locsds(idx_flat, loc_t, sds_t):
    def loc_map(k):
        return lambda i, idx: (0, idx[i * _IPS + k] // 128)

    grid_spec = pltpu.PrefetchScalarGridSpec(
        num_scalar_prefetch=1,
        grid=(_GSTEPS,),
        in_specs=(
            [pl.BlockSpec((_LOC_DIM, 128), loc_map(k)) for k in range(_IPS)]
            + [pl.BlockSpec((_LOC_DIM, 128), loc_map(k)) for k in range(_IPS)]
        ),
        out_specs=[
            pl.BlockSpec((_IPS, _LOC_DIM), lambda i, idx: (i, 0)),
            pl.BlockSpec((_IPS, _LOC_DIM), lambda i, idx: (i, 0)),
        ],
    )
    return pl.pallas_call(
        _locsds_body,
        grid_spec=grid_spec,
        out_shape=[
            jax.ShapeDtypeStruct((_N_ROWS, _LOC_DIM), jnp.float32),
            jax.ShapeDtypeStruct((_N_ROWS, _LOC_DIM), jnp.float32),
        ],
    )(idx_flat, *([loc_t] * _IPS), *([sds_t] * _IPS))


def kernel(drive_emb, mem_locations, mem_location_sds, mem_senses):
    idx, nf = _topk(drive_emb, mem_senses)
    idx_flat = idx.reshape(-1)
    sen = _gather_senses(idx_flat, mem_senses)
    loc, sds = _gather_locsds(idx_flat, mem_locations.T,
                              mem_location_sds.T)
    return (
        loc.reshape(1, _NUM_DRIVES, _K, _LOC_DIM),
        sds.reshape(1, _NUM_DRIVES, _K, _LOC_DIM),
        sen.reshape(1, _NUM_DRIVES, _K, _DIM),
        nf.reshape(1, _NUM_DRIVES),
    )


# speedup vs baseline: 1.0142x; 1.0142x over previous
import functools

import jax
import jax.numpy as jnp
from jax import lax
from jax.experimental import pallas as pl
from jax.experimental.pallas import tpu as pltpu
from jax.experimental.pallas import tpu_sc as plsc

_NUM_DRIVES = 64
_DIM = 128
_LOC_DIM = 64
_M = 100000
_K = 5
_THRESH = 0.1
_TILE = 10000
_GRID = _M // _TILE
def _topk_body(q_ref, s_ref, idx_ref, nf_ref, vals_s, idx_s):
    i = pl.program_id(0)
    neg = jnp.float32(-jnp.inf)
    _BIG = jnp.int32(2**31 - 1)

    q = q_ref[...]
    qn = q / (jnp.sqrt(jnp.sum(q * q, axis=1, keepdims=True)) + 1e-8)
    s = s_ref[...]
    nrm = jnp.sqrt(jnp.sum(s * s, axis=1, keepdims=True)) + 1e-8
    sn = (s / nrm).astype(jnp.bfloat16)
    qb = qn.astype(jnp.bfloat16)
    scores = lax.dot_general(qb, sn, (((1,), (1,)), ((), ())),
                             preferred_element_type=jnp.float32)

    bounds = (0, 4992, _TILE)
    halves = []
    for h in range(2):
        lo, hi = bounds[h], bounds[h + 1]
        cur = scores[:, lo:hi]
        col = lax.broadcasted_iota(jnp.int32, (_NUM_DRIVES, hi - lo), 1)
        tv, ti = [], []
        for _ in range(_K):
            m = jnp.max(cur, axis=1, keepdims=True)
            sel = jnp.min(jnp.where(cur == m, col, _BIG), axis=1,
                          keepdims=True)
            tv.append(m)
            ti.append(sel)
            cur = jnp.where(col == sel, neg, cur)
        off = i * _TILE + lo
        halves.append((jnp.concatenate(tv, axis=1),
                       jnp.concatenate(ti, axis=1) + off))

    @pl.when(i == 0)
    def _():
        vals_s[...] = jnp.full((_NUM_DRIVES, _K), neg, jnp.float32)
        idx_s[...] = jnp.full((_NUM_DRIVES, _K), 2**31 - 1, jnp.int32)

    cv = jnp.concatenate([vals_s[...], halves[0][0], halves[1][0]], axis=1)
    ci = jnp.concatenate([idx_s[...], halves[0][1], halves[1][1]], axis=1)
    ov, oi = [], []
    for _ in range(_K):
        m = jnp.max(cv, axis=1, keepdims=True)
        sel = jnp.min(jnp.where(cv == m, ci, _BIG), axis=1, keepdims=True)
        ov.append(m)
        oi.append(sel)
        cv = jnp.where(ci == sel, neg, cv)
    nv = jnp.concatenate(ov, axis=1)
    ni = jnp.concatenate(oi, axis=1)
    vals_s[...] = nv
    idx_s[...] = ni

    @pl.when(i == _GRID - 1)
    def _():
        idx_ref[...] = ni
        nf_ref[...] = jnp.sum((nv > _THRESH).astype(jnp.int32), axis=1,
                              keepdims=True)


def _topk(drive_emb, mem_senses):
    return pl.pallas_call(
        _topk_body,
        grid=(_GRID,),
        in_specs=[
            pl.BlockSpec((_NUM_DRIVES, _DIM), lambda i: (0, 0)),
            pl.BlockSpec((_TILE, _DIM), lambda i: (i, 0)),
        ],
        out_specs=[
            pl.BlockSpec((_NUM_DRIVES, _K), lambda i: (0, 0)),
            pl.BlockSpec((_NUM_DRIVES, 1), lambda i: (0, 0)),
        ],
        out_shape=[
            jax.ShapeDtypeStruct((_NUM_DRIVES, _K), jnp.int32),
            jax.ShapeDtypeStruct((_NUM_DRIVES, 1), jnp.int32),
        ],
        scratch_shapes=[
            pltpu.VMEM((_NUM_DRIVES, _K), jnp.float32),
            pltpu.VMEM((_NUM_DRIVES, _K), jnp.int32),
        ],
    )(drive_emb, mem_senses)


_N_ROWS = _NUM_DRIVES * _K
_N_WORKERS = 8
_ROWS_PER = _N_ROWS // _N_WORKERS


def _gather_senses(idx_flat, mem_senses):
    info = plsc.get_sparse_core_info()
    nc = info.num_cores

    mesh = plsc.VectorSubcoreMesh(core_axis_name="c", subcore_axis_name="s")

    @functools.partial(
        pl.kernel,
        mesh=mesh,
        out_type=jax.ShapeDtypeStruct((_N_ROWS, _DIM), jnp.float32),
        scratch_types=[
            pltpu.VMEM((_ROWS_PER,), jnp.int32),
            pltpu.VMEM((_ROWS_PER, _DIM), jnp.float32),
            pltpu.SemaphoreType.DMA,
        ],
    )
    def gk(idx_hbm, sen_hbm, out_sen, idx_v, sen_v, sem):
        wid = lax.axis_index("s") * nc + lax.axis_index("c")

        @pl.when(wid < _N_WORKERS)
        def _():
            base = pl.multiple_of(wid * _ROWS_PER, 8)
            pltpu.sync_copy(idx_hbm.at[pl.ds(base, _ROWS_PER)], idx_v)
            pltpu.async_copy(sen_hbm.at[idx_v], sen_v, sem).wait()
            pltpu.sync_copy(sen_v, out_sen.at[pl.ds(base, _ROWS_PER)])

    return gk(idx_flat, mem_senses)


_IPS = 16
_GSTEPS = _N_ROWS // _IPS


def _locsds_body(idx_ref, *refs):
    loc_blks = refs[:_IPS]
    sds_blks = refs[_IPS:2 * _IPS]
    out_loc, out_sds = refs[2 * _IPS], refs[2 * _IPS + 1]
    i = pl.program_id(0)
    dn = (((1,), (1,)), ((), ()))
    targets = jnp.concatenate(
        [jnp.full((1, 1), 128 * k, jnp.int32) + idx_ref[i * _IPS + k] % 128
         for k in range(_IPS)], axis=0)
    lanes = lax.broadcasted_iota(jnp.int32, (_IPS, 128 * _IPS), 1)
    e = (lanes == targets).astype(jnp.float32)
    loc_cat = jnp.concatenate([b[...] for b in loc_blks], axis=1)
    sds_cat = jnp.concatenate([b[...] for b in sds_blks], axis=1)
    out_loc[...] = lax.dot_general(e, loc_cat, dn,
                                   precision=lax.Precision.HIGHEST,
                                   preferred_element_type=jnp.float32)
    out_sds[...] = lax.dot_general(e, sds_cat, dn,
                                   precision=lax.Precision.HIGHEST,
                                   preferred_element_type=jnp.float32)


def _gather_locsds(idx_flat, loc_t, sds_t):
    def loc_map(k):
        return lambda i, idx: (0, idx[i * _IPS + k] // 128)

    grid_spec = pltpu.PrefetchScalarGridSpec(
        num_scalar_prefetch=1,
        grid=(_GSTEPS,),
        in_specs=(
            [pl.BlockSpec((_LOC_DIM, 128), loc_map(k)) for k in range(_IPS)]
            + [pl.BlockSpec((_LOC_DIM, 128), loc_map(k)) for k in range(_IPS)]
        ),
        out_specs=[
            pl.BlockSpec((_IPS, _LOC_DIM), lambda i, idx: (i, 0)),
            pl.BlockSpec((_IPS, _LOC_DIM), lambda i, idx: (i, 0)),
        ],
    )
    return pl.pallas_call(
        _locsds_body,
        grid_spec=grid_spec,
        out_shape=[
            jax.ShapeDtypeStruct((_N_ROWS, _LOC_DIM), jnp.float32),
            jax.ShapeDtypeStruct((_N_ROWS, _LOC_DIM), jnp.float32),
        ],
    )(idx_flat, *([loc_t] * _IPS), *([sds_t] * _IPS))


def kernel(drive_emb, mem_locations, mem_location_sds, mem_senses):
    idx, nf = _topk(drive_emb, mem_senses)
    idx_flat = idx.reshape(-1)
    sen = _gather_senses(idx_flat, mem_senses)
    loc, sds = _gather_locsds(idx_flat, mem_locations.T,
                              mem_location_sds.T)
    return (
        loc.reshape(1, _NUM_DRIVES, _K, _LOC_DIM),
        sds.reshape(1, _NUM_DRIVES, _K, _LOC_DIM),
        sen.reshape(1, _NUM_DRIVES, _K, _DIM),
        nf.reshape(1, _NUM_DRIVES),
    )


# speedup vs baseline: 1.0458x; 1.0311x over previous
import functools

import jax
import jax.numpy as jnp
from jax import lax
from jax.experimental import pallas as pl
from jax.experimental.pallas import tpu as pltpu
from jax.experimental.pallas import tpu_sc as plsc

_NUM_DRIVES = 64
_DIM = 128
_LOC_DIM = 64
_M = 100000
_K = 5
_THRESH = 0.1
_TILE = 10000
_GRID = _M // _TILE
def _topk_body(q_ref, s_ref, idx_ref, nf_ref, vals_s, idx_s):
    i = pl.program_id(0)
    neg = jnp.float32(-jnp.inf)
    _BIG = jnp.int32(2**31 - 1)

    q = q_ref[...]
    qn = q / (jnp.sqrt(jnp.sum(q * q, axis=1, keepdims=True)) + 1e-8)
    s = s_ref[...]
    nrm = jnp.sqrt(jnp.sum(s * s, axis=1, keepdims=True)) + 1e-8
    sn = (s / nrm).astype(jnp.bfloat16)
    qb = qn.astype(jnp.bfloat16)
    scores = lax.dot_general(qb, sn, (((1,), (1,)), ((), ())),
                             preferred_element_type=jnp.float32)

    bounds = (0, 4992, _TILE)
    halves = []
    for h in range(2):
        lo, hi = bounds[h], bounds[h + 1]
        cur = scores[:, lo:hi]
        col = lax.broadcasted_iota(jnp.int32, (_NUM_DRIVES, hi - lo), 1)
        tv, ti = [], []
        for _ in range(_K):
            m = jnp.max(cur, axis=1, keepdims=True)
            sel = jnp.min(jnp.where(cur == m, col, _BIG), axis=1,
                          keepdims=True)
            tv.append(m)
            ti.append(sel)
            cur = jnp.where(col == sel, neg, cur)
        off = i * _TILE + lo
        halves.append((jnp.concatenate(tv, axis=1),
                       jnp.concatenate(ti, axis=1) + off))

    @pl.when(i == 0)
    def _():
        vals_s[...] = jnp.full((_NUM_DRIVES, _K), neg, jnp.float32)
        idx_s[...] = jnp.full((_NUM_DRIVES, _K), 2**31 - 1, jnp.int32)

    cv = jnp.concatenate([vals_s[...], halves[0][0], halves[1][0]], axis=1)
    ci = jnp.concatenate([idx_s[...], halves[0][1], halves[1][1]], axis=1)
    ov, oi = [], []
    for _ in range(_K):
        m = jnp.max(cv, axis=1, keepdims=True)
        sel = jnp.min(jnp.where(cv == m, ci, _BIG), axis=1, keepdims=True)
        ov.append(m)
        oi.append(sel)
        cv = jnp.where(ci == sel, neg, cv)
    nv = jnp.concatenate(ov, axis=1)
    ni = jnp.concatenate(oi, axis=1)
    vals_s[...] = nv
    idx_s[...] = ni

    @pl.when(i == _GRID - 1)
    def _():
        idx_ref[...] = ni
        nf_ref[...] = jnp.sum((nv > _THRESH).astype(jnp.int32), axis=1,
                              keepdims=True)


def _topk(drive_emb, mem_senses):
    return pl.pallas_call(
        _topk_body,
        grid=(_GRID,),
        in_specs=[
            pl.BlockSpec((_NUM_DRIVES, _DIM), lambda i: (0, 0)),
            pl.BlockSpec((_TILE, _DIM), lambda i: (i, 0)),
        ],
        out_specs=[
            pl.BlockSpec((_NUM_DRIVES, _K), lambda i: (0, 0)),
            pl.BlockSpec((_NUM_DRIVES, 1), lambda i: (0, 0)),
        ],
        out_shape=[
            jax.ShapeDtypeStruct((_NUM_DRIVES, _K), jnp.int32),
            jax.ShapeDtypeStruct((_NUM_DRIVES, 1), jnp.int32),
        ],
        scratch_shapes=[
            pltpu.VMEM((_NUM_DRIVES, _K), jnp.float32),
            pltpu.VMEM((_NUM_DRIVES, _K), jnp.int32),
        ],
    )(drive_emb, mem_senses)


_N_ROWS = _NUM_DRIVES * _K
_N_WORKERS = 8
_ROWS_PER = _N_ROWS // _N_WORKERS


def _gather_senses(idx_flat, mem_senses):
    info = plsc.get_sparse_core_info()
    nc = info.num_cores

    mesh = plsc.VectorSubcoreMesh(core_axis_name="c", subcore_axis_name="s")

    @functools.partial(
        pl.kernel,
        mesh=mesh,
        out_type=jax.ShapeDtypeStruct((_N_ROWS, _DIM), jnp.float32),
        scratch_types=[
            pltpu.VMEM((_ROWS_PER,), jnp.int32),
            pltpu.VMEM((_ROWS_PER, _DIM), jnp.float32),
            pltpu.SemaphoreType.DMA,
        ],
    )
    def gk(idx_hbm, sen_hbm, out_sen, idx_v, sen_v, sem):
        wid = lax.axis_index("s") * nc + lax.axis_index("c")

        @pl.when(wid < _N_WORKERS)
        def _():
            base = pl.multiple_of(wid * _ROWS_PER, 8)
            pltpu.sync_copy(idx_hbm.at[pl.ds(base, _ROWS_PER)], idx_v)
            pltpu.async_copy(sen_hbm.at[idx_v], sen_v, sem).wait()
            pltpu.sync_copy(sen_v, out_sen.at[pl.ds(base, _ROWS_PER)])

    return gk(idx_flat, mem_senses)


_IPS = 16
_GSTEPS = _N_ROWS // _IPS


def _locsds_body(idx_ref, *refs):
    loc_blks = refs[:_IPS]
    sds_blks = refs[_IPS:2 * _IPS]
    out_loc, out_sds = refs[2 * _IPS], refs[2 * _IPS + 1]
    i = pl.program_id(0)
    dn = (((1,), (1,)), ((), ()))
    targets = jnp.concatenate(
        [jnp.full((1, 1), 128 * k, jnp.int32) + idx_ref[i * _IPS + k] % 128
         for k in range(_IPS)], axis=0)
    lanes = lax.broadcasted_iota(jnp.int32, (_IPS, 128 * _IPS), 1)
    e = (lanes == targets).astype(jnp.float32)
    loc_cat = jnp.concatenate([b[...] for b in loc_blks], axis=1)
    sds_cat = jnp.concatenate([b[...] for b in sds_blks], axis=1)
    out_loc[...] = lax.dot_general(e, loc_cat, dn,
                                   precision=lax.Precision.HIGHEST,
                                   preferred_element_type=jnp.float32)
    out_sds[...] = lax.dot_general(e, sds_cat, dn,
                                   precision=lax.Precision.HIGHEST,
                                   preferred_element_type=jnp.float32)


def _gather_locsds(idx_flat, loc_t, sds_t):
    def loc_map(k):
        return lambda i, idx: (0, idx[i * _IPS + k] // 128)

    grid_spec = pltpu.PrefetchScalarGridSpec(
        num_scalar_prefetch=1,
        grid=(_GSTEPS,),
        in_specs=(
            [pl.BlockSpec((_LOC_DIM, 128), loc_map(k)) for k in range(_IPS)]
            + [pl.BlockSpec((_LOC_DIM, 128), loc_map(k)) for k in range(_IPS)]
        ),
        out_specs=[
            pl.BlockSpec((_IPS, _LOC_DIM), lambda i, idx: (i, 0)),
            pl.BlockSpec((_IPS, _LOC_DIM), lambda i, idx: (i, 0)),
        ],
    )
    return pl.pallas_call(
        _locsds_body,
        grid_spec=grid_spec,
        out_shape=[
            jax.ShapeDtypeStruct((_N_ROWS, _LOC_DIM), jnp.float32),
            jax.ShapeDtypeStruct((_N_ROWS, _LOC_DIM), jnp.float32),
        ],
    )(idx_flat, *([loc_t] * _IPS), *([sds_t] * _IPS))


def kernel(drive_emb, mem_locations, mem_location_sds, mem_senses):
    idx, nf = _topk(drive_emb, mem_senses)
    idx_flat = idx.T.reshape(-1)
    sen = _gather_senses(idx_flat, mem_senses)
    loc, sds = _gather_locsds(idx_flat, mem_locations.T,
                              mem_location_sds.T)

    def _qkd(rows, d):
        return rows.reshape(_K, _NUM_DRIVES, d).transpose(1, 0, 2)[None]

    return (
        _qkd(loc, _LOC_DIM),
        _qkd(sds, _LOC_DIM),
        _qkd(sen, _DIM),
        nf.reshape(1, _NUM_DRIVES),
    )
